# baseline (speedup 1.0000x reference)
import functools

import jax
import jax.numpy as jnp
import numpy as np
from jax import lax
from jax.experimental import pallas as pl
from jax.experimental.pallas import tpu as pltpu
from jax.experimental.pallas import tpu_sc as plsc

L = 16


def _interleave_perm_half():
    lo = [32 * (p // 16) + p % 16 for p in range(64)]
    hi = [32 * (p // 16) + 16 + p % 16 for p in range(64)]
    return lo + hi


def _w_perms(d):
    half = _interleave_perm_half()
    full = list(range(2 * d))
    for t in (2, 3):
        for c in (0, 1):
            base = 256 * t + 128 * c
            full += [base + p for p in half]
    eperm = []
    for c in (0, 1):
        eperm += [128 * c + p for p in half]
    return np.array(full, np.int32), np.array(eperm, np.int32)


def _pack_bf16_u32(xf):
    u = lax.bitcast_convert_type(xf, jnp.uint32) + jnp.uint32(0x8000)
    return (u[:, 0:64] >> 16) | (u[:, 64:128] & jnp.uint32(0xFFFF0000))


def _node_proj_body(nf_ref, w_ref, b_ref, hq_ref, kvb_ref):
    x = jnp.dot(nf_ref[...], w_ref[...], preferred_element_type=jnp.float32)
    x = x + b_ref[...][None, :]
    for t in range(2):
        for c in range(2):
            hq_ref[2 * t + c] = x[:, 256 * t + 128 * c : 256 * t + 128 * c + 128]
    for c in range(2):
        kvb_ref[c, :, 0:64] = _pack_bf16_u32(x[:, 512 + 128 * c : 512 + 128 * c + 128])
        kvb_ref[c, :, 64:128] = _pack_bf16_u32(x[:, 768 + 128 * c : 768 + 128 * c + 128])


def _node_proj(nf, w, b, bn=512):
    n, d = nf.shape
    grid = (n // bn,)
    return pl.pallas_call(
        _node_proj_body,
        grid=grid,
        in_specs=[
            pl.BlockSpec((bn, d), lambda i: (i, 0)),
            pl.BlockSpec((d, 4 * d), lambda i: (0, 0)),
            pl.BlockSpec((4 * d,), lambda i: (0,)),
        ],
        out_specs=[
            pl.BlockSpec((4, bn, 128), lambda i: (0, i, 0)),
            pl.BlockSpec((2, bn, 128), lambda i: (0, i, 0)),
        ],
        out_shape=[
            jax.ShapeDtypeStruct((4, n, 128), jnp.float32),
            jax.ShapeDtypeStruct((2, n, 128), jnp.uint32),
        ],
    )(nf, w, b)


def _edge_proj_body(ef_ref, we_ref, be_ref, out_ref):
    y = jnp.dot(ef_ref[...], we_ref[...], preferred_element_type=jnp.float32)
    y = y + be_ref[...][None, :]
    out_ref[0] = _pack_bf16_u32(y[:, 0:128])
    out_ref[1] = _pack_bf16_u32(y[:, 128:256])


def _edge_proj(ef, we, be, be_blk=2000):
    e, de = ef.shape
    d = we.shape[1]
    grid = (e // be_blk,)
    return pl.pallas_call(
        _edge_proj_body,
        grid=grid,
        in_specs=[
            pl.BlockSpec((be_blk, de), lambda i: (i, 0)),
            pl.BlockSpec((de, d), lambda i: (0, 0)),
            pl.BlockSpec((d,), lambda i: (0,)),
        ],
        out_specs=pl.BlockSpec((2, be_blk, 64), lambda i: (0, i, 0)),
        out_shape=jax.ShapeDtypeStruct((2, e, 64), jnp.uint32),
    )(ef, we, be)


def _make_sc_kernel(npad, e, h):
    info = plsc.get_sparse_core_info()
    nc, ns = info.num_cores, info.num_subcores
    epw = e // ns
    B = 40
    IG = 10
    G = IG * B
    ngrp = epw // G
    nch = epw // B
    npw = npad // ns

    mesh = plsc.VectorSubcoreMesh(core_axis_name="c", subcore_axis_name="s")

    @functools.partial(
        pl.kernel,
        out_type=(
            jax.ShapeDtypeStruct((e, 2 * h), jnp.float32),
            jax.ShapeDtypeStruct((npad, 2 * h), jnp.float32),
        ),
        mesh=mesh,
        scratch_types=[
            pltpu.VMEM_SHARED((npad, h), jnp.float32),
            pltpu.VMEM((2, G), jnp.int32),
            pltpu.VMEM((G,), jnp.int32),
            pltpu.VMEM((G,), jnp.int32),
            [pltpu.VMEM((B,), jnp.int32) for _ in range(2)],
            [pltpu.VMEM((B, h), jnp.float32) for _ in range(2)],
            [pltpu.VMEM((B, h), jnp.uint32) for _ in range(2)],
            [pltpu.VMEM((B, h // 2), jnp.uint32) for _ in range(2)],
            [pltpu.VMEM((B, h), jnp.float32) for _ in range(2)],
            pltpu.SemaphoreType.DMA,
            [pltpu.SemaphoreType.DMA for _ in range(2)],
            [pltpu.SemaphoreType.DMA for _ in range(2)],
            [pltpu.SemaphoreType.DMA for _ in range(2)],
            [pltpu.SemaphoreType.DMA for _ in range(2)],
            [pltpu.SemaphoreType.DMA for _ in range(2)],
        ],
    )
    def sc_kernel(hq, kvt, efb, idx2, edges_out, nodes_out,
                  acc, stage, qig, kvig, rsc, qrows, kvrows, efrows,
                  mrows, sem_idx, sem_q, sem_kv, sem_ef, sem_we,
                  sem_ws):
        c = lax.axis_index("c")
        s = lax.axis_index("s")
        nvr = h // L

        pltpu.sync_copy(hq.at[pl.ds(c * npad + s * npw, npw)],
                        acc.at[pl.ds(s * npw, npw)])
        plsc.subcore_barrier()

        qbase = (2 + c) * npad
        kvbase = c * npad

        def adjust_group():
            def adj_body(i, _):
                sl = pl.ds(i * L, L)
                qig[sl] = stage[0, sl] + qbase
                kvig[sl] = stage[1, sl] + kvbase
                return 0
            lax.fori_loop(0, G // L, adj_body, 0)

        def prefetch_group(g):
            @pl.when(g < ngrp)
            def _():
                pltpu.make_async_copy(idx2.at[s, g], stage, sem_idx).start()

        def wait_stage():
            pltpu.make_async_copy(idx2.at[s, 0], stage, sem_idx).wait()

        def copy_rsc(b, k):
            o = k * B
            for st in (0, 16, B - L):
                rsc[b][pl.ds(st, L)] = qig[pl.ds(o + st, L)] - qbase

        def gather_descs(b, j):
            k = lax.rem(j, IG)
            e0 = s * epw + j * B
            return (
                pltpu.make_async_copy(hq.at[qig.at[pl.ds(k * B, B)]],
                                      qrows[b], sem_q[b]),
                pltpu.make_async_copy(kvt.at[kvig.at[pl.ds(k * B, B)]],
                                      kvrows[b], sem_kv[b]),
                pltpu.make_async_copy(efb.at[c, pl.ds(e0, B)],
                                      efrows[b], sem_ef[b]),
            )

        def issue_wb(b, j):
            e0 = s * epw + j * B
            pltpu.make_async_copy(
                qrows[b], edges_out.at[pl.ds(e0, B), pl.ds(c * h, h)],
                sem_we[b]).start()
            pltpu.async_copy(mrows[b], acc.at[rsc[b]], sem_ws[b], add=True)

        def wait_wb(b, j):
            e0 = s * epw + j * B
            pltpu.make_async_copy(
                qrows[b], edges_out.at[pl.ds(e0, B), pl.ds(c * h, h)],
                sem_we[b]).wait()
            pltpu.make_async_copy(mrows[b], acc.at[rsc[b]],
                                  sem_ws[b]).wait()

        def issue_gathers(b, j):
            for d in gather_descs(b, j):
                d.start()

        def wait_gathers(b, j):
            for d in gather_descs(b, j):
                d.wait()

        def compute(b):
            def row_body(r, _):
                ngl = nvr // 2

                def ld(ref, g0):
                    w = ref[r, pl.ds(g0 * L, L)]
                    lo = lax.bitcast_convert_type(w << 16, jnp.float32)
                    hi = lax.bitcast_convert_type(
                        w & jnp.uint32(0xFFFF0000), jnp.float32)
                    return (lo, hi)

                qs = [qrows[b][r, pl.ds(cv * L, L)] for cv in range(nvr)]
                ks = [ld(kvrows[b], g) for g in range(ngl)]
                vs = [ld(kvrows[b], ngl + g) for g in range(ngl)]
                es = [ld(efrows[b], g) for g in range(ngl)]
                evs = []
                for g in range(ngl):
                    for u in range(2):
                        evs.append(qs[2 * g + u] + ks[g][u] + es[g][u])
                etas = [1.0 / (1.0 + jnp.exp(-ev)) for ev in evs]
                for g in range(ngl):
                    for u in range(2):
                        qrows[b][r, pl.ds((2 * g + u) * L, L)] = evs[2 * g + u]
                for g in range(ngl):
                    for u in range(2):
                        mrows[b][r, pl.ds((2 * g + u) * L, L)] = (
                            etas[2 * g + u] * vs[g][u])
                return 0
            lax.fori_loop(0, B, row_body, 0)

        pltpu.sync_copy(idx2.at[s, 0], stage)
        adjust_group()
        prefetch_group(1)

        def pair_body(jj, _):
            for b in (0, 1):
                j = 2 * jj + b

                @pl.when(jj >= 1)
                def _():
                    wait_wb(b, j - 2)

                boundary = jnp.logical_and(jj > 0, lax.rem(jj, IG // 2) == 0)
                if b == 0:
                    @pl.when(boundary)
                    def _():
                        wait_gathers(1, j - 1)
                        wait_stage()
                        adjust_group()
                        prefetch_group(lax.div(j, IG) + 1)

                copy_rsc(b, lax.rem(j, IG))
                issue_gathers(b, j)

                if b == 0:
                    @pl.when(jnp.logical_and(j >= 1,
                                             jnp.logical_not(boundary)))
                    def _():
                        wait_gathers(1, j - 1)
                else:
                    wait_gathers(0, j - 1)

                @pl.when(j >= 1)
                def _():
                    compute(1 - b)
                    issue_wb(1 - b, j - 1)
            return 0
        lax.fori_loop(0, nch // 2, pair_body, 0)

        wait_gathers(1, nch - 1)
        compute(1)
        issue_wb(1, nch - 1)
        wait_wb(0, nch - 2)
        wait_wb(1, nch - 1)

        plsc.subcore_barrier()

        pltpu.sync_copy(
            acc.at[pl.ds(s * npw, npw)],
            nodes_out.at[pl.ds(s * npw, npw), pl.ds(c * h, h)])

    return sc_kernel


def kernel(node_features, senders, receivers, edge_features,
           W_kernel, W_bias, We_kernel, We_bias):
    n, d = node_features.shape
    e = senders.shape[0]
    h = d // 2
    npad = ((n + 16 * 80 - 1) // (16 * 80)) * (16 * 80)

    nf = node_features
    if npad != n:
        nf = jnp.pad(node_features, ((0, npad - n), (0, 0)))

    wperm, eperm = _w_perms(d)
    hq, kvb = _node_proj(nf, W_kernel[:, wperm], W_bias[wperm])
    efb = _edge_proj(edge_features, We_kernel[:, eperm], We_bias[eperm])

    hq_flat = hq.reshape(4 * npad, h)
    kvb_flat = kvb.reshape(2 * npad, h)

    ns, ig, bb = 16, 10, 40
    g = ig * bb
    ngrp = e // (ns * g)
    idx2 = jnp.stack(
        [receivers.astype(jnp.int32).reshape(ns, ngrp, g),
         senders.astype(jnp.int32).reshape(ns, ngrp, g)], axis=2)
    sc = _make_sc_kernel(npad, e, h)
    edges, nodes = sc(hq_flat, kvb_flat, efb, idx2)
    return (nodes[:n], edges)


# speedup vs baseline: 1.0821x; 1.0821x over previous
import functools

import jax
import jax.numpy as jnp
import numpy as np
from jax import lax
from jax.experimental import pallas as pl
from jax.experimental.pallas import tpu as pltpu
from jax.experimental.pallas import tpu_sc as plsc

L = 16


def _interleave_perm_half():
    lo = [32 * (p // 16) + p % 16 for p in range(64)]
    hi = [32 * (p // 16) + 16 + p % 16 for p in range(64)]
    return lo + hi


def _w_perms(d):
    half = _interleave_perm_half()
    full = list(range(2 * d))
    for t in (2, 3):
        for c in (0, 1):
            base = 256 * t + 128 * c
            full += [base + p for p in half]
    eperm = []
    for c in (0, 1):
        eperm += [128 * c + p for p in half]
    return np.array(full, np.int32), np.array(eperm, np.int32)


def _pack_bf16_u32(xf):
    u = lax.bitcast_convert_type(xf, jnp.uint32) + jnp.uint32(0x8000)
    return (u[:, 0:64] >> 16) | (u[:, 64:128] & jnp.uint32(0xFFFF0000))


def _node_proj_body(nf_ref, w_ref, b_ref, hq_ref, kvb_ref):
    x = jnp.dot(nf_ref[...], w_ref[...], preferred_element_type=jnp.float32)
    x = x + b_ref[...][None, :]
    for t in range(2):
        for c in range(2):
            hq_ref[2 * t + c] = x[:, 256 * t + 128 * c : 256 * t + 128 * c + 128]
    for c in range(2):
        kvb_ref[c, :, 0:64] = _pack_bf16_u32(x[:, 512 + 128 * c : 512 + 128 * c + 128])
        kvb_ref[c, :, 64:128] = _pack_bf16_u32(x[:, 768 + 128 * c : 768 + 128 * c + 128])


def _node_proj(nf, w, b, bn=1024):
    n, d = nf.shape
    grid = (n // bn,)
    return pl.pallas_call(
        _node_proj_body,
        grid=grid,
        in_specs=[
            pl.BlockSpec((bn, d), lambda i: (i, 0)),
            pl.BlockSpec((d, 4 * d), lambda i: (0, 0)),
            pl.BlockSpec((4 * d,), lambda i: (0,)),
        ],
        out_specs=[
            pl.BlockSpec((4, bn, 128), lambda i: (0, i, 0)),
            pl.BlockSpec((2, bn, 128), lambda i: (0, i, 0)),
        ],
        out_shape=[
            jax.ShapeDtypeStruct((4, n, 128), jnp.float32),
            jax.ShapeDtypeStruct((2, n, 128), jnp.uint32),
        ],
    )(nf, w, b)


def _edge_proj_body(ef_ref, we_ref, be_ref, out_ref):
    y = jnp.dot(ef_ref[...], we_ref[...], preferred_element_type=jnp.float32)
    y = y + be_ref[...][None, :]
    out_ref[0] = _pack_bf16_u32(y[:, 0:128])
    out_ref[1] = _pack_bf16_u32(y[:, 128:256])


def _edge_proj(ef, we, be, be_blk=8000):
    e, de = ef.shape
    d = we.shape[1]
    grid = (e // be_blk,)
    return pl.pallas_call(
        _edge_proj_body,
        grid=grid,
        in_specs=[
            pl.BlockSpec((be_blk, de), lambda i: (i, 0)),
            pl.BlockSpec((de, d), lambda i: (0, 0)),
            pl.BlockSpec((d,), lambda i: (0,)),
        ],
        out_specs=pl.BlockSpec((2, be_blk, 64), lambda i: (0, i, 0)),
        out_shape=jax.ShapeDtypeStruct((2, e, 64), jnp.uint32),
    )(ef, we, be)


def _make_sc_kernel(npad, e, h):
    info = plsc.get_sparse_core_info()
    nc, ns = info.num_cores, info.num_subcores
    epw = e // ns
    B = 40
    IG = 10
    G = IG * B
    ngrp = epw // G
    nch = epw // B
    npw = npad // ns

    mesh = plsc.VectorSubcoreMesh(core_axis_name="c", subcore_axis_name="s")

    @functools.partial(
        pl.kernel,
        out_type=(
            jax.ShapeDtypeStruct((e, 2 * h), jnp.float32),
            jax.ShapeDtypeStruct((npad, 2 * h), jnp.float32),
        ),
        mesh=mesh,
        scratch_types=[
            pltpu.VMEM_SHARED((npad, h), jnp.float32),
            pltpu.VMEM((2, G), jnp.int32),
            pltpu.VMEM((G,), jnp.int32),
            pltpu.VMEM((G,), jnp.int32),
            [pltpu.VMEM((B,), jnp.int32) for _ in range(2)],
            [pltpu.VMEM((B, h), jnp.float32) for _ in range(2)],
            [pltpu.VMEM((B, h), jnp.uint32) for _ in range(2)],
            [pltpu.VMEM((B, h // 2), jnp.uint32) for _ in range(2)],
            [pltpu.VMEM((B, h), jnp.float32) for _ in range(2)],
            pltpu.SemaphoreType.DMA,
            [pltpu.SemaphoreType.DMA for _ in range(2)],
            [pltpu.SemaphoreType.DMA for _ in range(2)],
            [pltpu.SemaphoreType.DMA for _ in range(2)],
            [pltpu.SemaphoreType.DMA for _ in range(2)],
            [pltpu.SemaphoreType.DMA for _ in range(2)],
        ],
    )
    def sc_kernel(hq, kvt, efb, idx2, edges_out, nodes_out,
                  acc, stage, qig, kvig, rsc, qrows, kvrows, efrows,
                  mrows, sem_idx, sem_q, sem_kv, sem_ef, sem_we,
                  sem_ws):
        c = lax.axis_index("c")
        s = lax.axis_index("s")
        nvr = h // L

        pltpu.sync_copy(hq.at[pl.ds(c * npad + s * npw, npw)],
                        acc.at[pl.ds(s * npw, npw)])
        plsc.subcore_barrier()

        qbase = (2 + c) * npad
        kvbase = c * npad

        def adjust_group():
            def adj_body(i, _):
                sl = pl.ds(i * L, L)
                qig[sl] = stage[0, sl] + qbase
                kvig[sl] = stage[1, sl] + kvbase
                return 0
            lax.fori_loop(0, G // L, adj_body, 0)

        def prefetch_group(g):
            @pl.when(g < ngrp)
            def _():
                pltpu.make_async_copy(idx2.at[s, g], stage, sem_idx).start()

        def wait_stage():
            pltpu.make_async_copy(idx2.at[s, 0], stage, sem_idx).wait()

        def copy_rsc(b, k):
            o = k * B
            for st in (0, 16, B - L):
                rsc[b][pl.ds(st, L)] = qig[pl.ds(o + st, L)] - qbase

        def gather_descs(b, j):
            k = lax.rem(j, IG)
            e0 = s * epw + j * B
            return (
                pltpu.make_async_copy(hq.at[qig.at[pl.ds(k * B, B)]],
                                      qrows[b], sem_q[b]),
                pltpu.make_async_copy(kvt.at[kvig.at[pl.ds(k * B, B)]],
                                      kvrows[b], sem_kv[b]),
                pltpu.make_async_copy(efb.at[c, pl.ds(e0, B)],
                                      efrows[b], sem_ef[b]),
            )

        def issue_wb(b, j):
            e0 = s * epw + j * B
            pltpu.make_async_copy(
                qrows[b], edges_out.at[pl.ds(e0, B), pl.ds(c * h, h)],
                sem_we[b]).start()
            pltpu.async_copy(mrows[b], acc.at[rsc[b]], sem_ws[b], add=True)

        def wait_wb(b, j):
            e0 = s * epw + j * B
            pltpu.make_async_copy(
                qrows[b], edges_out.at[pl.ds(e0, B), pl.ds(c * h, h)],
                sem_we[b]).wait()
            pltpu.make_async_copy(mrows[b], acc.at[rsc[b]],
                                  sem_ws[b]).wait()

        def issue_gathers(b, j):
            for d in gather_descs(b, j):
                d.start()

        def wait_gathers(b, j):
            for d in gather_descs(b, j):
                d.wait()

        def compute(b):
            def row_body(r, _):
                ngl = nvr // 2

                def ld(ref, g0):
                    w = ref[r, pl.ds(g0 * L, L)]
                    lo = lax.bitcast_convert_type(w << 16, jnp.float32)
                    hi = lax.bitcast_convert_type(
                        w & jnp.uint32(0xFFFF0000), jnp.float32)
                    return (lo, hi)

                qs = [qrows[b][r, pl.ds(cv * L, L)] for cv in range(nvr)]
                ks = [ld(kvrows[b], g) for g in range(ngl)]
                vs = [ld(kvrows[b], ngl + g) for g in range(ngl)]
                es = [ld(efrows[b], g) for g in range(ngl)]
                evs = []
                for g in range(ngl):
                    for u in range(2):
                        evs.append(qs[2 * g + u] + ks[g][u] + es[g][u])
                etas = [1.0 / (1.0 + jnp.exp(-ev)) for ev in evs]
                for g in range(ngl):
                    for u in range(2):
                        qrows[b][r, pl.ds((2 * g + u) * L, L)] = evs[2 * g + u]
                for g in range(ngl):
                    for u in range(2):
                        mrows[b][r, pl.ds((2 * g + u) * L, L)] = (
                            etas[2 * g + u] * vs[g][u])
                return 0
            lax.fori_loop(0, B, row_body, 0)

        pltpu.sync_copy(idx2.at[s, 0], stage)
        adjust_group()
        prefetch_group(1)

        def pair_body(jj, _):
            for b in (0, 1):
                j = 2 * jj + b

                @pl.when(jj >= 1)
                def _():
                    wait_wb(b, j - 2)

                boundary = jnp.logical_and(jj > 0, lax.rem(jj, IG // 2) == 0)
                if b == 0:
                    @pl.when(boundary)
                    def _():
                        wait_gathers(1, j - 1)
                        wait_stage()
                        adjust_group()
                        prefetch_group(lax.div(j, IG) + 1)

                copy_rsc(b, lax.rem(j, IG))
                issue_gathers(b, j)

                if b == 0:
                    @pl.when(jnp.logical_and(j >= 1,
                                             jnp.logical_not(boundary)))
                    def _():
                        wait_gathers(1, j - 1)
                else:
                    wait_gathers(0, j - 1)

                @pl.when(j >= 1)
                def _():
                    compute(1 - b)
                    issue_wb(1 - b, j - 1)
            return 0
        lax.fori_loop(0, nch // 2, pair_body, 0)

        wait_gathers(1, nch - 1)
        compute(1)
        issue_wb(1, nch - 1)
        wait_wb(0, nch - 2)
        wait_wb(1, nch - 1)

        plsc.subcore_barrier()

        pltpu.sync_copy(
            acc.at[pl.ds(s * npw, npw)],
            nodes_out.at[pl.ds(s * npw, npw), pl.ds(c * h, h)])

    return sc_kernel


def kernel(node_features, senders, receivers, edge_features,
           W_kernel, W_bias, We_kernel, We_bias):
    n, d = node_features.shape
    e = senders.shape[0]
    h = d // 2
    npad = ((n + 16 * 80 - 1) // (16 * 80)) * (16 * 80)

    nf = node_features
    if npad != n:
        nf = jnp.pad(node_features, ((0, npad - n), (0, 0)))

    wperm, eperm = _w_perms(d)
    hq, kvb = _node_proj(nf, W_kernel[:, wperm], W_bias[wperm])
    efb = _edge_proj(edge_features, We_kernel[:, eperm], We_bias[eperm])

    hq_flat = hq.reshape(4 * npad, h)
    kvb_flat = kvb.reshape(2 * npad, h)

    ns, ig, bb = 16, 10, 40
    g = ig * bb
    ngrp = e // (ns * g)
    idx2 = jnp.stack(
        [receivers.astype(jnp.int32).reshape(ns, ngrp, g),
         senders.astype(jnp.int32).reshape(ns, ngrp, g)], axis=2)
    sc = _make_sc_kernel(npad, e, h)
    edges, nodes = sc(hq_flat, kvb_flat, efb, idx2)
    return (nodes[:n], edges)
